# baseline (speedup 1.0000x reference)
import functools

import jax
import jax.numpy as jnp
from jax import lax
from jax.experimental import pallas as pl
from jax.experimental.pallas import tpu as pltpu
from jax.experimental.pallas import tpu_sc as plsc

_C = 64
_HC = _C // 2
_CK = 16
_BLANK = 63
_B = 16
_T = 1024
_NV = _T // 16
_YLEN = 1600
_YH = _YLEN // 2
_LN2 = 0.6931471805599453


def _mlog(v):
    bits = lax.bitcast_convert_type(v, jnp.int32)
    e = ((bits >> 23) & 0xFF) - 127
    mant = lax.bitcast_convert_type(
        (bits & 0x007FFFFF) | 0x3F800000, jnp.float32)
    big = mant > 1.5
    mant = jnp.where(big, mant * 0.5, mant)
    e = jnp.where(big, e + 1, e)
    z = (mant - 1.0) / (mant + 1.0)
    z2 = z * z
    p = z * (2.0 + z2 * (2.0 / 3.0 + z2 * (2.0 / 5.0 + z2 * (2.0 / 7.0
             + z2 * (2.0 / 9.0 + z2 * (2.0 / 11.0))))))
    return p + e.astype(jnp.float32) * _LN2


def _sc_body(x_hbm, y_hbm, tl_hbm, out_hbm,
             xv, bestb, idxb, otherb, pnk, pyk, lbuf, obuf, npad,
             spall,
             yv, tlv, sv, hist, yhist, nks, yks,
             sem0, sem1, ysem, tsem):
    c = lax.axis_index("c")
    s = lax.axis_index("s")
    b = s // 2 + 8 * c
    hc = s % 2

    lane = lax.iota(jnp.int32, 16)
    ones = jnp.ones((16,), jnp.float32)
    zeros = jnp.zeros((16,), jnp.float32)

    pltpu.async_copy(y_hbm, yv, ysem)
    pltpu.async_copy(tl_hbm, tlv, tsem)

    row0 = hc * _HC
    cp0 = pltpu.async_copy(x_hbm.at[b, pl.ds(row0, _CK), :], xv.at[0], sem0)
    cp1 = pltpu.async_copy(x_hbm.at[b, pl.ds(row0 + _CK, _CK), :], xv.at[1],
                           sem1)

    for i in range(_HC):
        hist[i, :] = zeros
    for i in range(_C):
        yhist[i, :] = zeros

    def _chains(buf, base, coff):
        pairs = []
        for k in range(4):
            c0 = k * 4
            best = buf[c0, pl.ds(base, 16)]
            bidx = jnp.full((16,), coff + c0, jnp.int32)
            for cc in range(c0 + 1, c0 + 4):
                v = buf[cc, pl.ds(base, 16)]
                m = v > best
                best = jnp.where(m, v, best)
                bidx = jnp.where(m, jnp.full((16,), coff + cc, jnp.int32),
                                 bidx)
            pairs.append((best, bidx))
        while len(pairs) > 1:
            nxt = []
            for lo, hi in zip(pairs[0::2], pairs[1::2]):
                m = hi[0] > lo[0]
                nxt.append((jnp.where(m, hi[0], lo[0]),
                            jnp.where(m, hi[1], lo[1])))
            pairs = nxt
        return pairs[0]

    cp0.wait()

    def c0_body(j, _):
        base = j * 16
        best, bidx = _chains(xv.at[0], base, 0)
        bestb[pl.ds(base, 16)] = best
        idxb[pl.ds(base, 16)] = bidx
        return 0
    lax.fori_loop(0, _NV, c0_body, 0)

    cp1.wait()

    def c1_body(j, _):
        base = j * 16
        best, bidx = _chains(xv.at[1], base, _CK)
        b0 = bestb[pl.ds(base, 16)]
        i0 = idxb[pl.ds(base, 16)]
        m = best > b0
        bestb[pl.ds(base, 16)] = jnp.where(m, best, b0)
        idxb[pl.ds(base, 16)] = jnp.where(m, bidx, i0)
        return 0
    lax.fori_loop(0, _NV, c1_body, 0)

    pltpu.sync_copy(bestb, spall.at[s, pl.ds(0, _T)])
    plsc.subcore_barrier()
    pltpu.sync_copy(spall.at[s ^ 1, pl.ds(0, _T)], otherb)

    low = jnp.full((16,), hc, jnp.int32) == 0

    def win_body(j, _):
        base = j * 16
        mine = bestb[pl.ds(base, 16)]
        oth = otherb[pl.ds(base, 16)]
        idx = idxb[pl.ds(base, 16)]
        win = (mine > oth) | ((mine == oth) & low)
        plsc.addupdate_scatter(hist, [idx, lane], ones, mask=win)
        return 0
    lax.fori_loop(0, _NV, win_body, 0)

    pltpu.make_async_copy(y_hbm, yv, ysem).wait()
    pltpu.make_async_copy(tl_hbm, tlv, tsem).wait()
    tl = tlv[...]
    incl = plsc.cumsum(tl)
    sv[...] = incl - tl
    bsplat = jnp.full((16,), b, jnp.int32)
    start = plsc.load_gather(sv, [bsplat])
    end = start + plsc.load_gather(tlv, [bsplat])
    yoff = hc * _YH

    def y_body(k, _):
        for u in range(5):
            t0 = yoff + (k * 5 + u) * 16
            yy = yv[pl.ds(t0, 16)]
            t = t0 + lane
            m = (t >= start) & (t < end)
            plsc.addupdate_scatter(yhist, [yy, lane], ones, mask=m)
        return 0
    lax.fori_loop(0, _YH // 80, y_body, 0)

    def _lane_reduce(src, dst, rows):
        for g in range(rows // 16):
            cls = g * 16 + lane
            acc = zeros
            for k in range(16):
                col = jnp.full((16,), k, jnp.int32)
                acc = acc + plsc.load_gather(src, [cls, col])
            dst[pl.ds(g * 16, 16)] = acc

    _lane_reduce(hist, nks, _HC)
    _lane_reduce(yhist, yks, _C)
    npad[pl.ds(0, 16)] = nks[pl.ds(0, 16)]
    npad[pl.ds(16, 16)] = nks[pl.ds(16, 16)]
    npad[pl.ds(32, 16)] = zeros
    npad[pl.ds(48, 16)] = zeros
    pltpu.sync_copy(npad, spall.at[s, pl.ds(_T, _C)])
    pltpu.sync_copy(yks, spall.at[s, pl.ds(_T + _C, _C)])
    plsc.subcore_barrier()

    @pl.when(hc == 0)
    def _loss():
        pltpu.sync_copy(spall.at[s ^ 1, pl.ds(_T, _C)], pnk)
        pltpu.sync_copy(spall.at[s ^ 1, pl.ds(_T + _C, _C)], pyk)

        nk_g, yk_g, mask_g = [], [], []
        accn = zeros
        accy = zeros
        for g in range(4):
            if g < 2:
                nk = nks[pl.ds(g * 16, 16)]
            else:
                nk = pnk[pl.ds((g - 2) * 16, 16)]
            yk = yks[pl.ds(g * 16, 16)] + pyk[pl.ds(g * 16, 16)]
            msk = yk != 0.0
            nk_g.append(nk)
            yk_g.append(yk)
            mask_g.append(msk)
            accn = accn + jnp.where(msk, nk, 0.0)
            accy = accy + yk
        blank_lane = lane == 15
        nk_blank = jnp.sum(jnp.where(blank_lane, nk_g[3], 0.0))
        yk_blank = jnp.sum(jnp.where(blank_lane, yk_g[3], 0.0))
        denom_n = jnp.sum(accn) - nk_blank
        denom_y = jnp.sum(accy) - yk_blank
        dnv = zeros + denom_n
        dyv = zeros + denom_y
        acc_kl = zeros
        for g in range(4):
            n_p = jnp.maximum(nk_g[g] / dnv, 1e-5)
            y_p = yk_g[g] / dyv
            m = (n_p + y_p) / 2.0
            t1 = n_p * _mlog(n_p / m)
            t2 = y_p * _mlog(y_p / m)
            acc_kl = acc_kl + jnp.where(mask_g[g], t1 + t2, 0.0)
        lsplat = zeros + jnp.sum(acc_kl)
        for g in range(4):
            lbuf[pl.ds(g * 16, 16)] = lsplat
        pltpu.sync_copy(lbuf, spall.at[s, pl.ds(_T + 2 * _C, _C)])

    plsc.subcore_barrier()

    @pl.when(s == 0)
    def _partial():
        acc = zeros
        for p in range(8):
            pltpu.sync_copy(spall.at[2 * p, pl.ds(_T + 2 * _C, _C)], lbuf)
            acc = acc + lbuf[pl.ds(0, 16)]
        obuf[...] = acc
        pltpu.sync_copy(obuf, out_hbm.at[c])


_sc_loss = functools.partial(
    pl.kernel,
    mesh=plsc.VectorSubcoreMesh(core_axis_name="c", subcore_axis_name="s"),
    compiler_params=pltpu.CompilerParams(needs_layout_passes=False),
    out_type=jax.ShapeDtypeStruct((2, 16), jnp.float32),
    scratch_types=[
        pltpu.VMEM((2, _CK, _T), jnp.float32),
        pltpu.VMEM((_T,), jnp.float32),
        pltpu.VMEM((_T,), jnp.int32),
        pltpu.VMEM((_T,), jnp.float32),
        pltpu.VMEM((_C,), jnp.float32),
        pltpu.VMEM((_C,), jnp.float32),
        pltpu.VMEM((_C,), jnp.float32),
        pltpu.VMEM((16,), jnp.float32),
        pltpu.VMEM((_C,), jnp.float32),
        pltpu.VMEM_SHARED((16, _T + 3 * _C), jnp.float32),
        pltpu.VMEM((_YLEN,), jnp.int32),
        pltpu.VMEM((16,), jnp.int32),
        pltpu.VMEM((16,), jnp.int32),
        pltpu.VMEM((_HC, 16), jnp.float32),
        pltpu.VMEM((_C, 16), jnp.float32),
        pltpu.VMEM((_HC,), jnp.float32),
        pltpu.VMEM((_C,), jnp.float32),
        pltpu.SemaphoreType.DMA,
        pltpu.SemaphoreType.DMA,
        pltpu.SemaphoreType.DMA,
        pltpu.SemaphoreType.DMA,
    ],
)(_sc_body)


def kernel(x, y, target_lengths):
    partials = _sc_loss(x, y, target_lengths.astype(jnp.int32))
    return (partials[0, 0] + partials[1, 0]) / jnp.float32(_B)


# speedup vs baseline: 1.1134x; 1.1134x over previous
import functools

import jax
import jax.numpy as jnp
from jax import lax
from jax.experimental import pallas as pl
from jax.experimental.pallas import tpu as pltpu
from jax.experimental.pallas import tpu_sc as plsc

_C = 64
_BLANK = 63
_B = 16
_T = 1024
_HALF = _T // 2
_W = 128
_NCH = _HALF // _W
_YLEN = 1600
_YV = _YLEN // 16


def _sc_body(x_hbm, y_hbm, tl_hbm, nk_out, yk_out,
             xv, yv, tlv, sv, hist, yhist, nks, yks,
             sem0, sem1, ysem, tsem):
    c = lax.axis_index("c")
    s = lax.axis_index("s")
    b = s
    h = c
    yduty = h == (b % 2)

    lane = lax.iota(jnp.int32, 16)
    ones = jnp.ones((16,), jnp.float32)
    zeros = jnp.zeros((16,), jnp.float32)
    sems = (sem0, sem1)

    @pl.when(yduty)
    def _y_start():
        pltpu.async_copy(y_hbm, yv, ysem)
        pltpu.async_copy(tl_hbm, tlv, tsem)

    col0 = h * _HALF
    cur = pltpu.async_copy(x_hbm.at[b, :, pl.ds(col0, _W)], xv.at[0], sem0)

    for i in range(_C):
        hist[i, :] = zeros

    def _process(buf):
        def t_body(j, _):
            base = j * 16
            pairs = []
            for k in range(8):
                c0 = k * 8
                best = buf[c0, pl.ds(base, 16)]
                bidx = jnp.full((16,), c0, jnp.int32)
                for cc in range(c0 + 1, c0 + 8):
                    v = buf[cc, pl.ds(base, 16)]
                    m = v > best
                    best = jnp.where(m, v, best)
                    bidx = jnp.where(m, jnp.full((16,), cc, jnp.int32), bidx)
                pairs.append((best, bidx))
            while len(pairs) > 1:
                nxt = []
                for lo, hi in zip(pairs[0::2], pairs[1::2]):
                    m = hi[0] > lo[0]
                    nxt.append((jnp.where(m, hi[0], lo[0]),
                                jnp.where(m, hi[1], lo[1])))
                pairs = nxt
            plsc.addupdate_scatter(hist, [pairs[0][1], lane], ones)
            return 0
        lax.fori_loop(0, _W // 16, t_body, 0)

    for ch in range(_NCH):
        nxt = None
        if ch + 1 < _NCH:
            nxt = pltpu.async_copy(
                x_hbm.at[b, :, pl.ds(col0 + (ch + 1) * _W, _W)],
                xv.at[(ch + 1) % 2], sems[(ch + 1) % 2])
        cur.wait()
        _process(xv.at[ch % 2])
        cur = nxt

    def _lane_reduce(src, dst):
        for g in range(_C // 16):
            cls = g * 16 + lane
            acc = zeros
            for k in range(16):
                col = jnp.full((16,), k, jnp.int32)
                acc = acc + plsc.load_gather(src, [cls, col])
            dst[pl.ds(g * 16, 16)] = acc

    _lane_reduce(hist, nks)
    pltpu.sync_copy(nks, nk_out.at[h, b])

    @pl.when(yduty)
    def _y_hist():
        for i in range(_C):
            yhist[i, :] = zeros
        pltpu.make_async_copy(y_hbm, yv, ysem).wait()
        pltpu.make_async_copy(tl_hbm, tlv, tsem).wait()
        tl = tlv[...]
        incl = plsc.cumsum(tl)
        sv[...] = incl - tl
        bsplat = jnp.full((16,), b, jnp.int32)
        start = plsc.load_gather(sv, [bsplat])
        end = start + plsc.load_gather(tlv, [bsplat])

        def y_body(k, _):
            for u in range(4):
                t0 = (k * 4 + u) * 16
                yy = yv[pl.ds(t0, 16)]
                t = t0 + lane
                m = (t >= start) & (t < end)
                plsc.addupdate_scatter(yhist, [yy, lane], ones, mask=m)
            return 0
        lax.fori_loop(0, _YV // 4, y_body, 0)

        _lane_reduce(yhist, yks)
        pltpu.sync_copy(yks, yk_out.at[b])


_sc_counts = functools.partial(
    pl.kernel,
    mesh=plsc.VectorSubcoreMesh(core_axis_name="c", subcore_axis_name="s"),
    compiler_params=pltpu.CompilerParams(needs_layout_passes=False),
    out_type=[
        jax.ShapeDtypeStruct((2, _B, _C), jnp.float32),
        jax.ShapeDtypeStruct((_B, _C), jnp.float32),
    ],
    scratch_types=[
        pltpu.VMEM((2, _C, _W), jnp.float32),
        pltpu.VMEM((_YLEN,), jnp.int32),
        pltpu.VMEM((16,), jnp.int32),
        pltpu.VMEM((16,), jnp.int32),
        pltpu.VMEM((_C, 16), jnp.float32),
        pltpu.VMEM((_C, 16), jnp.float32),
        pltpu.VMEM((_C,), jnp.float32),
        pltpu.VMEM((_C,), jnp.float32),
        pltpu.SemaphoreType.DMA,
        pltpu.SemaphoreType.DMA,
        pltpu.SemaphoreType.DMA,
        pltpu.SemaphoreType.DMA,
    ],
)(_sc_body)


def _tc_loss_body(nk_ref, yk_ref, out_ref):
    nk = nk_ref[0] + nk_ref[1]
    yk = yk_ref[...]
    mask = yk != 0.0
    denom_n = jnp.sum(jnp.where(mask, nk, 0.0), axis=1) - nk[:, _BLANK]
    denom_y = jnp.sum(yk, axis=1) - yk[:, _BLANK]
    n_p = jnp.clip(nk / denom_n[:, None], 1e-5)
    y_p = yk / denom_y[:, None]
    m = (n_p + y_p) / 2.0
    kl1 = jnp.sum(jnp.where(mask, n_p * jnp.log(n_p / m), 0.0), axis=1)
    kl2 = jnp.sum(jnp.where(mask, y_p * jnp.log(y_p / m), 0.0), axis=1)
    out_ref[...] = jnp.full((1, 1), 1.0, jnp.float32) * jnp.mean(kl1 + kl2)


def kernel(x, y, target_lengths):
    nk, yk = _sc_counts(x, y, target_lengths.astype(jnp.int32))
    loss = pl.pallas_call(
        _tc_loss_body,
        out_shape=jax.ShapeDtypeStruct((1, 1), jnp.float32),
    )(nk, yk)
    return loss[0, 0]
